# Initial kernel scaffold; baseline (speedup 1.0000x reference)
import jax
import jax.numpy as jnp
from jax.experimental import pallas as pl


def kernel(feat, edge_index, W, W2):
    raise NotImplementedError("write your pallas kernel here")


# speedup vs baseline: 12.7523x; 12.7523x over previous
import functools

import jax
import jax.numpy as jnp
from jax import lax
from jax.experimental import pallas as pl
from jax.experimental.pallas import tpu as pltpu
from jax.experimental.pallas import tpu_sc as plsc

_N = 10000
_E = 320000
_RANK = 32
_OUT = 64
_PW = 2 * _RANK

_NC = 2
_NS = 16
_NW = _NC * _NS
_EPW = _E // _NW
_NB = _N // 16
_CH = 80
_NCH = _EPW // _CH
_NP = 10240
_RPS = _NP // _NS


def _mesh():
    return plsc.VectorSubcoreMesh(
        core_axis_name="c", subcore_axis_name="s",
        num_cores=_NC, num_subcores=_NS)


@functools.partial(
    pl.kernel,
    out_type=(jax.ShapeDtypeStruct((_NW, _N), jnp.float32),
              jax.ShapeDtypeStruct((_NW, _N), jnp.float32)),
    mesh=_mesh(),
    scratch_types=[pltpu.VMEM((_EPW,), jnp.int32),
                   pltpu.VMEM((_EPW,), jnp.int32),
                   pltpu.VMEM((_N,), jnp.float32),
                   pltpu.VMEM((_N,), jnp.float32)],
    compiler_params=pltpu.CompilerParams(needs_layout_passes=False),
)
def _deg_kernel(src_hbm, dst_hbm, zero_hbm, os_hbm, od_hbm, srcv, dstv, hs, hd):
    cid = lax.axis_index("c")
    sid = lax.axis_index("s")
    w = sid * _NC + cid
    base = w * _EPW
    pltpu.sync_copy(src_hbm.at[pl.ds(base, _EPW)], srcv)
    pltpu.sync_copy(dst_hbm.at[pl.ds(base, _EPW)], dstv)
    pltpu.sync_copy(zero_hbm, hs)
    pltpu.sync_copy(zero_hbm, hd)
    ones = jnp.ones((16,), jnp.float32)

    def body(i, carry):
        s = srcv[pl.ds(i * 16, 16)]
        d = dstv[pl.ds(i * 16, 16)]
        plsc.addupdate_scatter(hs, [s], ones)
        plsc.addupdate_scatter(hd, [d], ones)
        return carry

    lax.fori_loop(0, _EPW // 16, body, 0)
    pltpu.sync_copy(hs, os_hbm.at[w])
    pltpu.sync_copy(hd, od_hbm.at[w])


@functools.partial(
    pl.kernel,
    out_type=jax.ShapeDtypeStruct((_NC, _NP, _PW), jnp.float32),
    mesh=_mesh(),
    scratch_types=[pltpu.VMEM((_NCH, _CH), jnp.int32),
                   pltpu.VMEM((_NCH, _CH), jnp.int32),
                   pltpu.VMEM((_CH, _PW), jnp.float32),
                   pltpu.VMEM_SHARED((_NP, _PW), jnp.float32),
                   pltpu.SemaphoreType.DMA],
    compiler_params=pltpu.CompilerParams(use_tc_tiling_on_sc=False),
)
def _agg_kernel(p_hbm, src_hbm, dst_hbm, zero_hbm, out_hbm,
                srcv, dstv, rows, acc, sem):
    cid = lax.axis_index("c")
    sid = lax.axis_index("s")
    w = sid * _NC + cid
    pltpu.sync_copy(src_hbm.at[w], srcv)
    pltpu.sync_copy(dst_hbm.at[w], dstv)
    pltpu.sync_copy(zero_hbm, acc.at[pl.ds(sid * _RPS, _RPS)])
    plsc.subcore_barrier()

    def body(j, carry):
        pltpu.async_copy(p_hbm.at[srcv.at[j]], rows, sem).wait()
        pltpu.sync_copy(rows, acc.at[dstv.at[j]], add=True)
        return carry

    lax.fori_loop(0, _NCH, body, 0)
    plsc.subcore_barrier()
    pltpu.sync_copy(acc.at[pl.ds(sid * _RPS, _RPS)],
                    out_hbm.at[cid].at[pl.ds(sid * _RPS, _RPS)])


def _payload_call(feat, W, deg_src):
    def body(f_ref, w_ref, d_ref, o_ref):
        g = jnp.dot(f_ref[...], w_ref[...], preferred_element_type=jnp.float32)
        nsrc = lax.rsqrt(jnp.maximum(d_ref[...], 1.0))
        h = jnp.tanh(g * nsrc)
        o_ref[...] = jnp.concatenate(
            [jnp.log(jnp.abs(h)), (h < 0).astype(jnp.float32)], axis=1)

    return pl.pallas_call(
        body, out_shape=jax.ShapeDtypeStruct((_N, _PW), jnp.float32),
    )(feat, W, deg_src)


def _final_call(parts, deg_dst, W2):
    def body(s_ref, d_ref, w2_ref, o_ref):
        s = s_ref[0, :_N] + s_ref[1, :_N]
        sum_log = s[:, :_RANK]
        neg_cnt = s[:, _RANK:]
        sign = 1.0 - 2.0 * jnp.mod(neg_cnt, 2.0)
        r = sign * jnp.exp(sum_log)
        dd = d_ref[...]
        r = jnp.where(dd > 0.0, r, 0.0)
        r = r * lax.rsqrt(jnp.maximum(dd, 1.0))
        o_ref[...] = jnp.dot(r, w2_ref[...], preferred_element_type=jnp.float32)

    return pl.pallas_call(
        body, out_shape=jax.ShapeDtypeStruct((_N, _OUT), jnp.float32),
    )(parts, deg_dst, W2)


def kernel(feat, edge_index, W, W2):
    src = edge_index[0]
    dst = edge_index[1]
    zero_h = jnp.zeros((_N,), jnp.float32)
    hs, hd = _deg_kernel(src, dst, zero_h)
    deg_src = hs.sum(axis=0).reshape(_N, 1)
    deg_dst = hd.sum(axis=0).reshape(_N, 1)
    P = _payload_call(feat, W, deg_src)
    src3 = src.reshape(_NW, _NCH, _CH)
    dst3 = dst.reshape(_NW, _NCH, _CH)
    zero_r = jnp.zeros((_RPS, _PW), jnp.float32)
    parts = _agg_kernel(P, src3, dst3, zero_r)
    return _final_call(parts, deg_dst, W2)


# speedup vs baseline: 13.3154x; 1.0442x over previous
import functools

import jax
import jax.numpy as jnp
from jax import lax
from jax.experimental import pallas as pl
from jax.experimental.pallas import tpu as pltpu
from jax.experimental.pallas import tpu_sc as plsc

_N = 10000
_E = 320000
_RANK = 32
_OUT = 64
_PW = 2 * _RANK

_NC = 2
_NS = 16
_NW = _NC * _NS
_EPW = _E // _NW
_NB = _N // 16
_CH = 128
_NCH = 79
_EPWP = _NCH * _CH
_NP = 10240
_RPS = _NP // _NS


def _mesh():
    return plsc.VectorSubcoreMesh(
        core_axis_name="c", subcore_axis_name="s",
        num_cores=_NC, num_subcores=_NS)


@functools.partial(
    pl.kernel,
    out_type=(jax.ShapeDtypeStruct((_NW, _N), jnp.float32),
              jax.ShapeDtypeStruct((_NW, _N), jnp.float32)),
    mesh=_mesh(),
    scratch_types=[pltpu.VMEM((_EPW,), jnp.int32),
                   pltpu.VMEM((_EPW,), jnp.int32),
                   pltpu.VMEM((_N,), jnp.float32),
                   pltpu.VMEM((_N,), jnp.float32)],
    compiler_params=pltpu.CompilerParams(needs_layout_passes=False),
)
def _deg_kernel(src_hbm, dst_hbm, zero_hbm, os_hbm, od_hbm, srcv, dstv, hs, hd):
    cid = lax.axis_index("c")
    sid = lax.axis_index("s")
    w = sid * _NC + cid
    base = w * _EPW
    pltpu.sync_copy(src_hbm.at[pl.ds(base, _EPW)], srcv)
    pltpu.sync_copy(dst_hbm.at[pl.ds(base, _EPW)], dstv)
    pltpu.sync_copy(zero_hbm, hs)
    pltpu.sync_copy(zero_hbm, hd)
    ones = jnp.ones((16,), jnp.float32)

    def body(i, carry):
        s = srcv[pl.ds(i * 16, 16)]
        d = dstv[pl.ds(i * 16, 16)]
        plsc.addupdate_scatter(hs, [s], ones)
        plsc.addupdate_scatter(hd, [d], ones)
        return carry

    lax.fori_loop(0, _EPW // 16, body, 0)
    pltpu.sync_copy(hs, os_hbm.at[w])
    pltpu.sync_copy(hd, od_hbm.at[w])


@functools.partial(
    pl.kernel,
    out_type=jax.ShapeDtypeStruct((_NC, _NP, _PW), jnp.float32),
    mesh=_mesh(),
    scratch_types=[pltpu.VMEM((_NCH, _CH), jnp.int32),
                   pltpu.VMEM((_NCH, _CH), jnp.int32),
                   pltpu.VMEM((_CH, _PW), jnp.float32),
                   pltpu.VMEM((_CH, _PW), jnp.float32),
                   pltpu.VMEM_SHARED((_NP, _PW), jnp.float32),
                   pltpu.SemaphoreType.DMA,
                   pltpu.SemaphoreType.DMA],
    compiler_params=pltpu.CompilerParams(use_tc_tiling_on_sc=False),
)
def _agg_kernel(p_hbm, src_hbm, dst_hbm, zero_hbm, out_hbm,
                srcv, dstv, rows0, rows1, acc, sem0, sem1):
    cid = lax.axis_index("c")
    sid = lax.axis_index("s")
    w = sid * _NC + cid
    pltpu.sync_copy(src_hbm.at[w], srcv)
    pltpu.sync_copy(dst_hbm.at[w], dstv)
    pltpu.sync_copy(zero_hbm, acc.at[pl.ds(sid * _RPS, _RPS)])
    plsc.subcore_barrier()

    pltpu.async_copy(p_hbm.at[srcv.at[0]], rows0, sem0)

    def body(t, carry):
        j = 2 * t
        pltpu.async_copy(p_hbm.at[srcv.at[j + 1]], rows1, sem1)
        pltpu.make_async_copy(p_hbm.at[srcv.at[j]], rows0, sem0).wait()
        pltpu.sync_copy(rows0, acc.at[dstv.at[j]], add=True)
        pltpu.async_copy(p_hbm.at[srcv.at[j + 2]], rows0, sem0)
        pltpu.make_async_copy(p_hbm.at[srcv.at[j + 1]], rows1, sem1).wait()
        pltpu.sync_copy(rows1, acc.at[dstv.at[j + 1]], add=True)
        return carry

    lax.fori_loop(0, (_NCH - 1) // 2, body, 0)
    pltpu.make_async_copy(p_hbm.at[srcv.at[_NCH - 1]], rows0, sem0).wait()
    pltpu.sync_copy(rows0, acc.at[dstv.at[_NCH - 1]], add=True)
    plsc.subcore_barrier()
    pltpu.sync_copy(acc.at[pl.ds(sid * _RPS, _RPS)],
                    out_hbm.at[cid].at[pl.ds(sid * _RPS, _RPS)])


def _payload_call(feat, W, deg_src):
    def body(f_ref, w_ref, d_ref, o_ref):
        g = jnp.dot(f_ref[...], w_ref[...], preferred_element_type=jnp.float32)
        nsrc = lax.rsqrt(jnp.maximum(d_ref[...], 1.0))
        h = jnp.tanh(g * nsrc)
        o_ref[...] = jnp.concatenate(
            [jnp.log(jnp.abs(h)), (h < 0).astype(jnp.float32)], axis=1)

    return pl.pallas_call(
        body, out_shape=jax.ShapeDtypeStruct((_N, _PW), jnp.float32),
    )(feat, W, deg_src)


def _final_call(parts, deg_dst, W2):
    def body(s_ref, d_ref, w2_ref, o_ref):
        s = s_ref[0, :_N] + s_ref[1, :_N]
        sum_log = s[:, :_RANK]
        neg_cnt = s[:, _RANK:]
        sign = 1.0 - 2.0 * jnp.mod(neg_cnt, 2.0)
        r = sign * jnp.exp(sum_log)
        dd = d_ref[...]
        r = jnp.where(dd > 0.0, r, 0.0)
        r = r * lax.rsqrt(jnp.maximum(dd, 1.0))
        o_ref[...] = jnp.dot(r, w2_ref[...], preferred_element_type=jnp.float32)

    return pl.pallas_call(
        body, out_shape=jax.ShapeDtypeStruct((_N, _OUT), jnp.float32),
    )(parts, deg_dst, W2)


def kernel(feat, edge_index, W, W2):
    src = edge_index[0]
    dst = edge_index[1]
    zero_h = jnp.zeros((_N,), jnp.float32)
    hs, hd = _deg_kernel(src, dst, zero_h)
    deg_src = hs.sum(axis=0).reshape(_N, 1)
    deg_dst = hd.sum(axis=0).reshape(_N, 1)
    P = _payload_call(feat, W, deg_src)
    pad = _EPWP - _EPW
    src3 = jnp.concatenate(
        [src.reshape(_NW, _EPW), jnp.zeros((_NW, pad), jnp.int32)],
        axis=1).reshape(_NW, _NCH, _CH)
    dst3 = jnp.concatenate(
        [dst.reshape(_NW, _EPW), jnp.full((_NW, pad), _N, jnp.int32)],
        axis=1).reshape(_NW, _NCH, _CH)
    zero_r = jnp.zeros((_RPS, _PW), jnp.float32)
    parts = _agg_kernel(P, src3, dst3, zero_r)
    return _final_call(parts, deg_dst, W2)
